# baseline (speedup 1.0000x reference)
import functools

import jax
import jax.numpy as jnp
from jax import lax
from jax.experimental import pallas as pl
from jax.experimental.pallas import tpu as pltpu
from jax.experimental.pallas import tpu_sc as plsc

_NUM_CORES = 2
_NUM_SUBCORES = 16
_NUM_WORKERS = _NUM_CORES * _NUM_SUBCORES
_CHUNK = 128
_LANES = 16


@functools.lru_cache(maxsize=None)
def _build(B, V2, D):
    b_per_w = B // _NUM_WORKERS
    half = b_per_w // 2
    mesh = plsc.VectorSubcoreMesh(core_axis_name="c", subcore_axis_name="s")

    @functools.partial(
        pl.kernel,
        mesh=mesh,
        out_type=jax.ShapeDtypeStruct((B, D), jnp.float32),
        compiler_params=pltpu.CompilerParams(
            use_tc_tiling_on_sc=True, needs_layout_passes=False
        ),
        scratch_types=[
            pltpu.VMEM((b_per_w,), jnp.int32),
            pltpu.VMEM((b_per_w,), jnp.int32),
            pltpu.VMEM((half, 2 * D), jnp.float32),
            pltpu.VMEM((b_per_w, D), jnp.float32),
            pltpu.SemaphoreType.DMA,
            pltpu.SemaphoreType.DMA,
        ],
    )
    def emb(idx_hbm, table2_hbm, out_hbm, idx_v, j_v, pair_v, stag_v, sem_i, sem_g):
        wid = lax.axis_index("s") * _NUM_CORES + lax.axis_index("c")
        base = wid * b_per_w
        pltpu.async_copy(idx_hbm.at[wid], idx_v, sem_i).wait()

        def jbody(g, _):
            wv = idx_v[pl.ds(g * _LANES, _LANES)]
            j_v[pl.ds(g * _LANES, _LANES)] = lax.shift_right_logical(wv, 1)
            return ()

        lax.fori_loop(0, b_per_w // _LANES, jbody, ())

        lane_iota = lax.iota(jnp.int32, _LANES)
        for h in range(2):
            cps = [
                pltpu.async_copy(
                    table2_hbm.at[j_v.at[pl.ds(h * half + c * _CHUNK, _CHUNK)]],
                    pair_v.at[pl.ds(c * _CHUNK, _CHUNK), :],
                    sem_g,
                )
                for c in range(half // _CHUNK)
            ]
            for cp in cps:
                cp.wait()

            def ebody(g, _, h=h):
                wbase = h * half + g * _LANES
                wv = idx_v[pl.ds(wbase, _LANES)]
                parv = lax.mul(lax.bitwise_and(wv, 1), D)
                for lane in range(_LANES):
                    colbase = parv[lane]
                    rows = jnp.full((_LANES,), g * _LANES + lane, jnp.int32)
                    for cidx in range(D // _LANES):
                        cols = colbase + cidx * _LANES + lane_iota
                        x = plsc.load_gather(pair_v, [rows, cols])
                        plsc.store_scatter(
                            stag_v,
                            [
                                jnp.full((_LANES,), wbase + lane, jnp.int32),
                                cidx * _LANES + lane_iota,
                            ],
                            x,
                        )
                return ()

            lax.fori_loop(0, half // _LANES, ebody, ())

        pltpu.sync_copy(stag_v, out_hbm.at[pl.ds(base, b_per_w), :])

    return emb


def kernel(word, table):
    (B,) = word.shape
    V, D = table.shape
    idx = word.astype(jnp.int32).reshape(_NUM_WORKERS, B // _NUM_WORKERS)
    table2 = table.reshape(V // 2, 2 * D)
    return _build(B, V // 2, D)(idx, table2)


# speedup vs baseline: 3.2029x; 3.2029x over previous
import functools

import jax
import jax.numpy as jnp
from jax import lax
from jax.experimental import pallas as pl
from jax.experimental.pallas import tpu as pltpu
from jax.experimental.pallas import tpu_sc as plsc

_NUM_CORES = 2
_NUM_SUBCORES = 16
_NUM_WORKERS = _NUM_CORES * _NUM_SUBCORES
_LANES = 16
_BATCH = 4
_NBANK = 2


@functools.lru_cache(maxsize=None)
def _build(B, V, D):
    b_per_w = B // _NUM_WORKERS
    n_groups = b_per_w // _LANES
    nslots = _BATCH * _NBANK
    mesh = plsc.VectorSubcoreMesh(core_axis_name="c", subcore_axis_name="s")

    @functools.partial(
        pl.kernel,
        mesh=mesh,
        out_type=jax.ShapeDtypeStruct((D, B), jnp.float32),
        compiler_params=pltpu.CompilerParams(
            use_tc_tiling_on_sc=True, needs_layout_passes=False
        ),
        scratch_types=[
            pltpu.VMEM((b_per_w,), jnp.int32),
            pltpu.VMEM((nslots, D, 128), jnp.float32),
            pltpu.VMEM((D, b_per_w), jnp.float32),
            pltpu.SemaphoreType.DMA,
            pltpu.SemaphoreType.DMA,
        ],
    )
    def emb(idx_hbm, tableT_hbm, outT_hbm, idx_v, tiles_v, stag_v, sem_i, sem_g):
        wid = lax.axis_index("s") * _NUM_CORES + lax.axis_index("c")
        base = wid * b_per_w
        pltpu.async_copy(idx_hbm.at[wid], idx_v, sem_i).wait()

        lane_iota = lax.iota(jnp.int32, _LANES)

        def issue(jb_scalar, slot):
            col0 = pl.multiple_of(jb_scalar * 128, 128)
            pltpu.async_copy(
                tableT_hbm.at[:, pl.ds(col0, 128)],
                tiles_v.at[slot],
                sem_g,
            )

        def wait_one(slot):
            pltpu.make_async_copy(
                tableT_hbm.at[:, pl.ds(0, 128)], tiles_v.at[slot], sem_g
            ).wait()

        def extract(c_scalar, slot, kloc):
            for q in range(D // _LANES):
                rows = q * _LANES + lane_iota
                x = plsc.load_gather(
                    tiles_v.at[slot],
                    [rows, jnp.full((_LANES,), c_scalar, jnp.int32)],
                )
                plsc.store_scatter(
                    stag_v,
                    [rows, jnp.full((_LANES,), kloc, jnp.int32)],
                    x,
                )

        def gbody(g, _):
            wv = idx_v[pl.ds(g * _LANES, _LANES)]
            wv2 = idx_v[pl.ds(lax.rem((g + 1), n_groups) * _LANES, _LANES)]
            jbv = lax.shift_right_logical(wv, 7)
            jbv2 = lax.shift_right_logical(wv2, 7)
            cv = lax.bitwise_and(wv, 127)
            gbase = g * _LANES

            for t in range(_LANES):
                tn = t + _BATCH
                nslot = tn % nslots
                if tn < _LANES:
                    jb_next = jbv[tn]
                else:
                    jb_next = jbv2[tn - _LANES]
                issue(jb_next, nslot)
                wait_one(t % nslots)
                extract(cv[t], t % nslots, gbase + t)
            return ()

        wv0 = idx_v[pl.ds(0, _LANES)]
        jbv0 = lax.shift_right_logical(wv0, 7)
        for t in range(_BATCH):
            issue(jbv0[t], t)

        lax.fori_loop(0, n_groups, gbody, ())

        for t in range(_BATCH):
            wait_one(t % nslots)

        pltpu.sync_copy(stag_v, outT_hbm.at[:, pl.ds(base, b_per_w)])

    return emb


def kernel(word, table):
    (B,) = word.shape
    V, D = table.shape
    idx = word.astype(jnp.int32).reshape(_NUM_WORKERS, B // _NUM_WORKERS)
    outT = _build(B, V, D)(idx, table.T)
    return outT.T
